# baseline (speedup 1.0000x reference)
import functools

import jax
import jax.numpy as jnp
from jax import lax
from jax.experimental import pallas as pl
from jax.experimental.pallas import tpu as pltpu
from jax.experimental.pallas import tpu_sc as plsc

SEQ = 200
DIM = 128
BATCH = 4096
NC = 2
NS = 16
NW = NC * NS
ROWS = BATCH * SEQ
ROWS_PER_W = ROWS // NW
NCHUNK = ROWS_PER_W // SEQ
NBUF = 3


def _emb_body(idx_hbm, tok_hbm, pos_hbm, out_hbm,
              g0, g1, g2, i0, i1, i2, pos_v,
              gs0, gs1, gs2, is0, is1, is2, ss0, ss1, ss2):
    G = [g0, g1, g2]
    I = [i0, i1, i2]
    GS = [gs0, gs1, gs2]
    IS = [is0, is1, is2]
    SS = [ss0, ss1, ss2]

    wid = lax.axis_index("s") * NC + lax.axis_index("c")
    base = wid * ROWS_PER_W

    def idx_issue(c, slot):
        pltpu.async_copy(idx_hbm.at[pl.ds(base + c * SEQ, SEQ)],
                         I[slot], IS[slot])

    def idx_wait(slot):
        pltpu.make_async_copy(idx_hbm.at[pl.ds(0, SEQ)],
                              I[slot], IS[slot]).wait()

    def gather_issue(slot):
        pltpu.async_copy(tok_hbm.at[I[slot]], G[slot], GS[slot])

    def gather_wait(slot):
        pltpu.make_async_copy(tok_hbm.at[pl.ds(0, SEQ)],
                              G[slot], GS[slot]).wait()

    def store_issue(c, slot):
        pltpu.async_copy(G[slot], out_hbm.at[pl.ds(base + c * SEQ, SEQ)],
                         SS[slot])

    def store_wait(slot):
        pltpu.make_async_copy(G[slot], out_hbm.at[pl.ds(0, SEQ)],
                              SS[slot]).wait()

    def add_chunk(slot):
        gbuf = G[slot]

        def p_body(p, carry):
            for d in range(DIM // 16):
                sl = pl.ds(d * 16, 16)
                gbuf[p, sl] = gbuf[p, sl] + pos_v[p, sl]
            return carry

        if True:
            return
        lax.fori_loop(0, SEQ, p_body, 0, unroll=False)

    def substep(c, slot, do_idx, do_store_wait, do_next_gather):
        nb = (slot + 1) % NBUF
        gather_wait(slot)
        if do_idx:
            idx_issue(c + NBUF, slot)
        if do_next_gather:
            if do_store_wait:
                store_wait(nb)
            idx_wait(nb)
            gather_issue(nb)
        add_chunk(slot)
        store_issue(c, slot)

    pltpu.sync_copy(pos_hbm, pos_v)
    for b in range(NBUF):
        idx_issue(b, b)
    idx_wait(0)
    gather_issue(0)
    substep(0, 0, True, False, True)
    substep(1, 1, True, False, True)
    substep(2, 2, True, True, True)

    def outer(t, carry):
        for b in range(NBUF):
            substep(t * NBUF + b, b, True, True, True)
        return carry

    lax.fori_loop(1, 40, outer, 0, unroll=False)

    for c in range(120, NCHUNK):
        substep(c, c % NBUF, c + NBUF < NCHUNK, True, c + 1 < NCHUNK)
    store_wait((NCHUNK - 2) % NBUF)
    store_wait((NCHUNK - 1) % NBUF)


def kernel(inputs, token_table, pos_table):
    idx_flat = inputs.reshape(ROWS).astype(jnp.int32)
    mesh = plsc.VectorSubcoreMesh(core_axis_name="c", subcore_axis_name="s")
    k = functools.partial(
        pl.kernel,
        out_type=jax.ShapeDtypeStruct((ROWS, DIM), jnp.float32),
        mesh=mesh,
        scratch_types=(
            [pltpu.VMEM((SEQ, DIM), jnp.float32) for _ in range(NBUF)]
            + [pltpu.VMEM((SEQ,), jnp.int32) for _ in range(NBUF)]
            + [pltpu.VMEM((SEQ, DIM), jnp.float32)]
            + [pltpu.SemaphoreType.DMA for _ in range(3 * NBUF)]
        ),
    )(_emb_body)
    out = k(idx_flat, token_table, pos_table)
    return out.reshape(BATCH, SEQ, DIM)


# speedup vs baseline: 1.0025x; 1.0025x over previous
import functools

import jax
import jax.numpy as jnp
from jax import lax
from jax.experimental import pallas as pl
from jax.experimental.pallas import tpu as pltpu
from jax.experimental.pallas import tpu_sc as plsc

SEQ = 200
DIM = 128
BATCH = 4096
NC = 2
NS = 16
NW = NC * NS
ROWS = BATCH * SEQ
ROWS_PER_W = ROWS // NW
NCHUNK = ROWS_PER_W // SEQ
NBUF = 3


def _emb_body(idx_hbm, tok_hbm, pos_hbm, out_hbm,
              g0, g1, g2, idx_v, pos_v,
              gs0, gs1, gs2, ss0, ss1, ss2):
    G = [g0, g1, g2]
    GS = [gs0, gs1, gs2]
    SS = [ss0, ss1, ss2]

    wid = lax.axis_index("s") * NC + lax.axis_index("c")
    base = wid * ROWS_PER_W

    def gather_issue(c, slot):
        pltpu.async_copy(tok_hbm.at[idx_v.at[pl.ds(c * SEQ, SEQ)]],
                         G[slot], GS[slot])

    def gather_wait(slot):
        pltpu.make_async_copy(tok_hbm.at[pl.ds(0, SEQ)],
                              G[slot], GS[slot]).wait()

    def store_issue(c, slot):
        pltpu.async_copy(G[slot], out_hbm.at[pl.ds(base + c * SEQ, SEQ)],
                         SS[slot])

    def store_wait(slot):
        pltpu.make_async_copy(G[slot], out_hbm.at[pl.ds(0, SEQ)],
                              SS[slot]).wait()

    def add_chunk(slot):
        gbuf = G[slot]

        def p_body(p, carry):
            for d in range(DIM // 16):
                sl = pl.ds(d * 16, 16)
                gbuf[p, sl] = gbuf[p, sl] + pos_v[p, sl]
            return carry

        lax.fori_loop(0, SEQ, p_body, 0, unroll=False)

    def substep(c, slot, do_store_wait, do_next_gather):
        nb = (slot + 1) % NBUF
        gather_wait(slot)
        if do_next_gather:
            if do_store_wait:
                store_wait(nb)
            gather_issue(c + 1, nb)
        add_chunk(slot)
        store_issue(c, slot)

    pltpu.sync_copy(idx_hbm.at[pl.ds(base, ROWS_PER_W)], idx_v)
    pltpu.sync_copy(pos_hbm, pos_v)
    gather_issue(0, 0)
    substep(0, 0, False, True)
    substep(1, 1, False, True)
    substep(2, 2, True, True)

    def outer(t, carry):
        for b in range(NBUF):
            substep(t * NBUF + b, b, True, True)
        return carry

    lax.fori_loop(1, 40, outer, 0, unroll=False)

    for c in range(120, NCHUNK):
        substep(c, c % NBUF, True, c + 1 < NCHUNK)
    store_wait((NCHUNK - 2) % NBUF)
    store_wait((NCHUNK - 1) % NBUF)


def kernel(inputs, token_table, pos_table):
    idx_flat = inputs.reshape(ROWS).astype(jnp.int32)
    mesh = plsc.VectorSubcoreMesh(core_axis_name="c", subcore_axis_name="s")
    k = functools.partial(
        pl.kernel,
        out_type=jax.ShapeDtypeStruct((ROWS, DIM), jnp.float32),
        mesh=mesh,
        scratch_types=(
            [pltpu.VMEM((SEQ, DIM), jnp.float32) for _ in range(NBUF)]
            + [pltpu.VMEM((ROWS_PER_W,), jnp.int32)]
            + [pltpu.VMEM((SEQ, DIM), jnp.float32)]
            + [pltpu.SemaphoreType.DMA for _ in range(2 * NBUF)]
        ),
    )(_emb_body)
    out = k(idx_flat, token_table, pos_table)
    return out.reshape(BATCH, SEQ, DIM)
